# baseline (speedup 1.0000x reference)
import jax
import jax.numpy as jnp
from jax.experimental import pallas as pl
from jax.experimental.pallas import tpu as pltpu

_QB = 512
_N_QUERY = 1024
_N_TRAIN = 16384
_D = 16


def _nn_block_kernel(x_ref, yt_ref, label_ref, out_ref):
    x = x_ref[...]
    yt = yt_ref[...]
    ynorm = jnp.sum(yt * yt, axis=0, keepdims=True)
    f32 = jnp.float32
    bf16 = jnp.bfloat16
    m2x = -2.0 * x
    m2x_hi = m2x.astype(bf16)
    m2x_lo = (m2x - m2x_hi.astype(f32)).astype(bf16)
    yt_hi = yt.astype(bf16)
    yt_lo = (yt - yt_hi.astype(f32)).astype(bf16)
    yn_hi = ynorm.astype(bf16)
    yn_r = ynorm - yn_hi.astype(f32)
    yn_mid = yn_r.astype(bf16)
    yn_lo = (yn_r - yn_mid.astype(f32)).astype(bf16)
    ones = jnp.ones((_QB, 1), bf16)
    xk = jnp.concatenate(
        [m2x_hi, m2x_hi, m2x_lo, ones, ones, ones], axis=1)
    ytk = jnp.concatenate(
        [yt_hi, yt_lo, yt_hi, yn_hi, yn_mid, yn_lo], axis=0)
    dist = jax.lax.dot_general(
        xk, ytk, (((1,), (0,)), ((), ())),
        preferred_element_type=jnp.float32)
    first_idx = jnp.argmin(dist, axis=1)[:, None]
    lab = label_ref[...]
    hi = first_idx >> 7
    lo = first_idx & 127
    iota_c = jax.lax.broadcasted_iota(jnp.int32, (_QB, 128), 1)
    oh_hi = (iota_c == hi).astype(bf16)
    oh_lo = (iota_c == lo).astype(f32)
    lab_hi = lab.astype(bf16)
    lab_r = lab - lab_hi.astype(f32)
    lab_mid = lab_r.astype(bf16)
    lab_lo = (lab_r - lab_mid.astype(f32)).astype(bf16)
    oh3 = jnp.concatenate([oh_hi, oh_hi, oh_hi], axis=1)
    lab3 = jnp.concatenate([lab_hi, lab_mid, lab_lo], axis=0)
    rows = jax.lax.dot_general(
        oh3, lab3, (((1,), (0,)), ((), ())),
        preferred_element_type=jnp.float32)
    out = jnp.sum(rows * oh_lo, axis=1)
    out_ref[...] = out.reshape(1, 1, _QB)


def kernel(x, train_pts, train_label):
    yt = train_pts.T
    labels = train_label.reshape(128, 128)
    n_blocks = _N_QUERY // _QB
    out = pl.pallas_call(
        _nn_block_kernel,
        grid=(n_blocks,),
        in_specs=[
            pl.BlockSpec((_QB, _D), lambda i: (i, 0)),
            pl.BlockSpec((_D, _N_TRAIN), lambda i: (0, 0)),
            pl.BlockSpec((128, 128), lambda i: (0, 0)),
        ],
        out_specs=pl.BlockSpec((1, 1, _QB), lambda i: (i, 0, 0)),
        out_shape=jax.ShapeDtypeStruct((n_blocks, 1, _QB), jnp.float32),
        compiler_params=pltpu.CompilerParams(
            dimension_semantics=("parallel",)),
    )(x, yt, labels)
    return out.reshape(_N_QUERY)


# speedup vs baseline: 1.0106x; 1.0106x over previous
import jax
import jax.numpy as jnp
from jax.experimental import pallas as pl
from jax.experimental.pallas import tpu as pltpu

_QB = 512
_N_QUERY = 1024
_N_TRAIN = 16384
_D = 16


def _nn_block_kernel(x_ref, yt_ref, label_ref, out_ref):
    x = x_ref[...]
    yt = yt_ref[...]
    ynorm = jnp.sum(yt * yt, axis=0, keepdims=True)
    f32 = jnp.float32
    bf16 = jnp.bfloat16
    m2x = -2.0 * x
    m2x_hi = m2x.astype(bf16)
    m2x_lo = (m2x - m2x_hi.astype(f32)).astype(bf16)
    yt_hi = yt.astype(bf16)
    yt_lo = (yt - yt_hi.astype(f32)).astype(bf16)
    yn_hi = ynorm.astype(bf16)
    yn_r = ynorm - yn_hi.astype(f32)
    yn_mid = yn_r.astype(bf16)
    yn_lo = (yn_r - yn_mid.astype(f32)).astype(bf16)
    ones = jnp.ones((_QB, 1), bf16)
    xk = jnp.concatenate(
        [m2x_hi, m2x_hi, m2x_lo, ones, ones, ones], axis=1)
    ytk = jnp.concatenate(
        [yt_hi, yt_lo, yt_hi, yn_hi, yn_mid, yn_lo], axis=0)
    dist = jax.lax.dot_general(
        xk, ytk, (((1,), (0,)), ((), ())),
        preferred_element_type=jnp.float32)
    first_idx = jnp.argmin(dist, axis=1)[:, None]
    lab = label_ref[...]
    hi = first_idx >> 7
    lo = first_idx & 127
    iota_c = jax.lax.broadcasted_iota(jnp.int32, (_QB, 128), 1)
    oh_hi = (iota_c == hi).astype(bf16)
    oh_lo = (iota_c == lo).astype(f32)
    lab_hi = lab.astype(bf16)
    lab_r = lab - lab_hi.astype(f32)
    lab_mid = lab_r.astype(bf16)
    lab_lo = (lab_r - lab_mid.astype(f32)).astype(bf16)
    oh3 = jnp.concatenate([oh_hi, oh_hi, oh_hi], axis=1)
    lab3 = jnp.concatenate([lab_hi, lab_mid, lab_lo], axis=0)
    rows = jax.lax.dot_general(
        oh3, lab3, (((1,), (0,)), ((), ())),
        preferred_element_type=jnp.float32)
    out = jnp.sum(rows * oh_lo, axis=1)
    out_ref[...] = out.reshape(1, 1, _QB)


def kernel(x, train_pts, train_label):
    yt = train_pts.T
    labels = train_label.reshape(128, 128)
    n_blocks = _N_QUERY // _QB
    out = pl.pallas_call(
        _nn_block_kernel,
        grid=(n_blocks,),
        in_specs=[
            pl.BlockSpec((_QB, _D), lambda i: (i, 0)),
            pl.BlockSpec((_D, _N_TRAIN), lambda i: (0, 0)),
            pl.BlockSpec((128, 128), lambda i: (0, 0)),
        ],
        out_specs=pl.BlockSpec((1, 1, _QB), lambda i: (i, 0, 0)),
        out_shape=jax.ShapeDtypeStruct((n_blocks, 1, _QB), jnp.float32),
    )(x, yt, labels)
    return out.reshape(_N_QUERY)
